# baseline (speedup 1.0000x reference)
import functools

import jax
import jax.numpy as jnp
from jax import lax
from jax.experimental import pallas as pl
from jax.experimental.pallas import tpu as pltpu
from jax.experimental.pallas import tpu_sc as plsc

VOCAB = 8192
DIM = 1024
DFF = 4096
TOK = 4096
THRESH = 0.8
IDXPAD = TOK + 16

_NC, _NS = 2, 16
_NW = _NC * _NS

_sc_mesh = lambda: plsc.VectorSubcoreMesh(
    core_axis_name="c", subcore_axis_name="s")


def _wid():
    return lax.axis_index("s") * _NC + lax.axis_index("c")


_EMB_CH = 32


def _emb_gather_body(x_hbm, emb_hbm, out_hbm, idx_v, rows_v, sem):
    w = _wid()
    rows_per_w = TOK // _NW
    for k in range(rows_per_w // _EMB_CH):
        base = w * rows_per_w + k * _EMB_CH
        pltpu.sync_copy(x_hbm.at[pl.ds(base, _EMB_CH)], idx_v)
        pltpu.async_copy(emb_hbm.at[idx_v], rows_v, sem).wait()
        pltpu.sync_copy(rows_v, out_hbm.at[pl.ds(base, _EMB_CH)])


def _emb_gather(x_flat, emb):
    return pl.kernel(
        _emb_gather_body,
        out_type=jax.ShapeDtypeStruct((TOK, DIM), jnp.float32),
        mesh=_sc_mesh(),
        compiler_params=pltpu.CompilerParams(needs_layout_passes=False),
        scratch_types=[
            pltpu.VMEM((_EMB_CH,), jnp.int32),
            pltpu.VMEM((_EMB_CH, DIM), jnp.float32),
            pltpu.SemaphoreType.DMA,
        ],
    )(x_flat, emb)


_BLK0 = 256
_CH = 1024


def _mlp0_body(h_ref, w1_ref, w2_ref, wout_ref, g_ref, b_ref,
               logits_ref, hout_ref, mask_ref,
               w1b_v, w2b_v, woutb_v, stage_v, sem_w):

    @pl.when(pl.program_id(0) == 0)
    def _():
        for c in range(DFF // _CH):
            cp = pltpu.make_async_copy(
                w1_ref.at[:, pl.ds(c * _CH, _CH)],
                stage_v.at[:DIM], sem_w)
            cp.start()
            cp.wait()
            w1b_v[:, pl.ds(c * _CH, _CH)] = (
                stage_v[:DIM].astype(jnp.bfloat16))
        for c in range(DFF // _CH):
            cp = pltpu.make_async_copy(
                w2_ref.at[pl.ds(c * _CH, _CH)], stage_v, sem_w)
            cp.start()
            cp.wait()
            w2b_v[pl.ds(c * _CH, _CH)] = stage_v[...].astype(jnp.bfloat16)
        for c in range(VOCAB // _CH):
            cp = pltpu.make_async_copy(
                wout_ref.at[pl.ds(c * _CH, _CH)], stage_v, sem_w)
            cp.start()
            cp.wait()
            woutb_v[pl.ds(c * _CH, _CH)] = stage_v[...].astype(jnp.bfloat16)

    h = h_ref[...]
    mean = jnp.mean(h, axis=-1, keepdims=True)
    var = jnp.mean((h - mean) ** 2, axis=-1, keepdims=True)
    ln = (h - mean) / jnp.sqrt(var + 1e-5) * g_ref[...] + b_ref[...]
    mid = jax.nn.gelu(jnp.dot(ln.astype(jnp.bfloat16), w1b_v[...],
                              preferred_element_type=jnp.float32))
    hout = h + jnp.dot(mid.astype(jnp.bfloat16), w2b_v[...],
                       preferred_element_type=jnp.float32)
    hout_ref[...] = hout
    num = jnp.sum(h * hout, axis=-1, keepdims=True)
    den = (jnp.sqrt(jnp.sum(h * h, axis=-1, keepdims=True))
           * jnp.sqrt(jnp.sum(hout * hout, axis=-1, keepdims=True)) + 1e-8)
    cos = num / den
    mask_ref[...] = (cos < THRESH).astype(jnp.int32)
    logits_ref[...] = lax.dot_general(
        hout.astype(jnp.bfloat16), woutb_v[...],
        dimension_numbers=(((1,), (1,)), ((), ())),
        preferred_element_type=jnp.float32)


def _mlp0(h, W1, W2, Wout, g, b):
    return pl.pallas_call(
        _mlp0_body,
        grid=(TOK // _BLK0,),
        in_specs=[
            pl.BlockSpec((_BLK0, DIM), lambda i: (i, 0)),
            pl.BlockSpec(memory_space=pl.ANY),
            pl.BlockSpec(memory_space=pl.ANY),
            pl.BlockSpec(memory_space=pl.ANY),
            pl.BlockSpec((1, DIM), lambda i: (0, 0)),
            pl.BlockSpec((1, DIM), lambda i: (0, 0)),
        ],
        out_specs=[
            pl.BlockSpec((_BLK0, VOCAB), lambda i: (i, 0)),
            pl.BlockSpec((_BLK0, DIM), lambda i: (i, 0)),
            pl.BlockSpec((_BLK0, 1), lambda i: (i, 0)),
        ],
        out_shape=[
            jax.ShapeDtypeStruct((TOK, VOCAB), jnp.float32),
            jax.ShapeDtypeStruct((TOK, DIM), jnp.float32),
            jax.ShapeDtypeStruct((TOK, 1), jnp.int32),
        ],
        scratch_shapes=[
            pltpu.VMEM((DIM, DFF), jnp.bfloat16),
            pltpu.VMEM((DFF, DIM), jnp.bfloat16),
            pltpu.VMEM((VOCAB, DIM), jnp.bfloat16),
            pltpu.VMEM((_CH, DIM), jnp.float32),
            pltpu.SemaphoreType.DMA,
        ],
        compiler_params=pltpu.CompilerParams(
            dimension_semantics=("arbitrary",),
            vmem_limit_bytes=63 * 1024 * 1024),
    )(h, W1, W2, Wout, g, b)


def _compact_body(mask_hbm, idx_hbm, cnt_hbm, mask_v, idx_v, cnt_v):
    @pl.when(_wid() == 0)
    def _():
        pltpu.sync_copy(mask_hbm, mask_v)

        def step(i, off):
            mv = mask_v[pl.ds(i * 16, 16)]
            ids = lax.iota(jnp.int32, 16) + i * 16
            cnt = jnp.sum(mv, axis=0)
            plsc.store_compressed(idx_v.at[pl.ds(off, 16)], ids, mask=mv != 0)
            return off + cnt

        count = lax.fori_loop(0, TOK // 16, step, jnp.int32(0))
        idx_v[pl.ds(count, 16)] = jnp.zeros((16,), jnp.int32)
        cnt_v[...] = jnp.full((16,), count, dtype=jnp.int32)
        pltpu.sync_copy(idx_v, idx_hbm)
        pltpu.sync_copy(cnt_v, cnt_hbm)


def _compact(mask_flat):
    return pl.kernel(
        _compact_body,
        out_type=[
            jax.ShapeDtypeStruct((IDXPAD,), jnp.int32),
            jax.ShapeDtypeStruct((16,), jnp.int32),
        ],
        mesh=_sc_mesh(),
        compiler_params=pltpu.CompilerParams(needs_layout_passes=False),
        scratch_types=[
            pltpu.VMEM((TOK,), jnp.int32),
            pltpu.VMEM((IDXPAD,), jnp.int32),
            pltpu.VMEM((16,), jnp.int32),
        ],
    )(mask_flat)


def _gather_rows_body(idx_hbm, cnt_hbm, src_hbm, out_hbm,
                      cnt_v, idx_v, rows_v, sem):
    w = _wid()
    pltpu.sync_copy(cnt_hbm, cnt_v)
    count = jnp.max(cnt_v[...], axis=0)
    nchunks = (count + 15) // 16
    n_mine = jnp.maximum(0, (nchunks - w + _NW - 1) // _NW)

    def step(k, _):
        base = (w + k * _NW) * 16
        pltpu.sync_copy(idx_hbm.at[pl.ds(base, 16)], idx_v)
        pltpu.async_copy(src_hbm.at[idx_v], rows_v, sem).wait()
        pltpu.sync_copy(rows_v, out_hbm.at[pl.ds(base, 16)])
        return 0

    lax.fori_loop(0, n_mine, step, 0)


def _gather_rows(idx, cnt, src):
    return pl.kernel(
        _gather_rows_body,
        out_type=jax.ShapeDtypeStruct((TOK, DIM), jnp.float32),
        mesh=_sc_mesh(),
        compiler_params=pltpu.CompilerParams(needs_layout_passes=False),
        scratch_types=[
            pltpu.VMEM((16,), jnp.int32),
            pltpu.VMEM((16,), jnp.int32),
            pltpu.VMEM((16, DIM), jnp.float32),
            pltpu.SemaphoreType.DMA,
        ],
    )(idx, cnt, src)


_BLK1 = 128
_CH5 = 1024


def _mlp1_body(lg0_ref, hc_ref, w1_ref, w2_ref, wout_ref, g_ref, b_ref,
               cnt_ref, idx_ref, lg_ref,
               w1_v, w2_v, woutb_v, stage_v, hb_v, lrow_v,
               sem_w, sem_h, sem_s):
    del lg0_ref
    count = cnt_ref[0]
    nblk = (count + _BLK1 - 1) // _BLK1

    @pl.when(nblk > 0)
    def _():
        cw1 = pltpu.make_async_copy(w1_ref, w1_v, sem_w)
        cw1.start()
        cw2 = pltpu.make_async_copy(w2_ref, w2_v, sem_w)
        cw2.start()
        cw1.wait()
        cw2.wait()
        for c in range(VOCAB // _CH5):
            cp = pltpu.make_async_copy(
                wout_ref.at[pl.ds(c * _CH5, _CH5)], stage_v, sem_w)
            cp.start()
            cp.wait()
            woutb_v[pl.ds(c * _CH5, _CH5)] = stage_v[...].astype(
                jnp.bfloat16)
        g = g_ref[...]
        b = b_ref[...]

        def blk(bi, _):
            base = bi * _BLK1
            ch = pltpu.make_async_copy(hc_ref.at[pl.ds(base, _BLK1)], hb_v,
                                       sem_h)
            ch.start()
            ch.wait()
            h = hb_v[...]
            mean = jnp.mean(h, axis=-1, keepdims=True)
            var = jnp.mean((h - mean) ** 2, axis=-1, keepdims=True)
            ln = (h - mean) / jnp.sqrt(var + 1e-5) * g + b
            mid = jax.nn.gelu(jnp.dot(ln.astype(jnp.bfloat16),
                                      w1_v[...].astype(jnp.bfloat16),
                                      preferred_element_type=jnp.float32))
            hout = h + jnp.dot(mid.astype(jnp.bfloat16),
                               w2_v[...].astype(jnp.bfloat16),
                               preferred_element_type=jnp.float32)
            lrow_v[...] = lax.dot_general(
                hout.astype(jnp.bfloat16), woutb_v[...],
                dimension_numbers=(((1,), (1,)), ((), ())),
                preferred_element_type=jnp.float32)
            rows = jnp.minimum(count - base, _BLK1)

            def row(r, _):
                dst = idx_ref[base + r]
                c = pltpu.make_async_copy(lrow_v.at[pl.ds(r, 1)],
                                          lg_ref.at[pl.ds(dst, 1)],
                                          sem_s)
                c.start()
                c.wait()
                return 0

            lax.fori_loop(0, rows, row, 0)
            return 0

        lax.fori_loop(0, nblk, blk, 0)


def _mlp1_scatter(logits0, h_c, W1, W2, Wout, g, b, cnt, idx):
    return pl.pallas_call(
        _mlp1_body,
        in_specs=[
            pl.BlockSpec(memory_space=pl.ANY),
            pl.BlockSpec(memory_space=pl.ANY),
            pl.BlockSpec(memory_space=pl.ANY),
            pl.BlockSpec(memory_space=pl.ANY),
            pl.BlockSpec(memory_space=pl.ANY),
            pl.BlockSpec(memory_space=pltpu.VMEM),
            pl.BlockSpec(memory_space=pltpu.VMEM),
            pl.BlockSpec(memory_space=pltpu.SMEM),
            pl.BlockSpec(memory_space=pltpu.SMEM),
        ],
        out_specs=pl.BlockSpec(memory_space=pl.ANY),
        out_shape=jax.ShapeDtypeStruct((TOK, VOCAB), jnp.float32),
        scratch_shapes=[
            pltpu.VMEM((DIM, DFF), jnp.float32),
            pltpu.VMEM((DFF, DIM), jnp.float32),
            pltpu.VMEM((VOCAB, DIM), jnp.bfloat16),
            pltpu.VMEM((_CH5, DIM), jnp.float32),
            pltpu.VMEM((_BLK1, DIM), jnp.float32),
            pltpu.VMEM((_BLK1, VOCAB), jnp.float32),
            pltpu.SemaphoreType.DMA,
            pltpu.SemaphoreType.DMA,
            pltpu.SemaphoreType.DMA,
        ],
        input_output_aliases={0: 0},
        compiler_params=pltpu.CompilerParams(
            vmem_limit_bytes=63 * 1024 * 1024),
    )(logits0, h_c, W1, W2, Wout, g, b, cnt, idx)


def kernel(x, emb, W_out, W1_0, W2_0, g0, b0, W1_1, W2_1, g1, b1):
    batch, seq = x.shape
    x_flat = x.reshape(-1).astype(jnp.int32)

    h = _emb_gather(x_flat, emb)
    logits0, h_out0, mask_col = _mlp0(h, W1_0, W2_0, W_out,
                                      g0.reshape(1, DIM),
                                      b0.reshape(1, DIM))
    mask_flat = mask_col.reshape(TOK)
    idx, cntv = _compact(mask_flat)
    h_c = _gather_rows(idx, cntv, h_out0)
    logits = _mlp1_scatter(logits0, h_c, W1_1, W2_1, W_out,
                           g1.reshape(1, DIM), b1.reshape(1, DIM),
                           cntv, idx)

    count1 = cntv[0]
    exit_counts = jnp.stack([TOK - count1, count1]).astype(jnp.int32)
    shallow_ratio = exit_counts[0] / (batch * seq)
    cumulative_layers = jnp.arange(1, 3)
    total_layers = jnp.sum(exit_counts * cumulative_layers)
    compute_cost = total_layers / ((batch * seq) * 2)
    return (logits.reshape(batch, seq, VOCAB), exit_counts,
            shallow_ratio, compute_cost)


# speedup vs baseline: 1.0926x; 1.0926x over previous
import functools

import jax
import jax.numpy as jnp
from jax import lax
from jax.experimental import pallas as pl
from jax.experimental.pallas import tpu as pltpu
from jax.experimental.pallas import tpu_sc as plsc

VOCAB = 8192
DIM = 1024
DFF = 4096
TOK = 4096
THRESH = 0.8
IDXPAD = TOK + 16

_NC, _NS = 2, 16
_NW = _NC * _NS

_sc_mesh = lambda: plsc.VectorSubcoreMesh(
    core_axis_name="c", subcore_axis_name="s")


def _wid():
    return lax.axis_index("s") * _NC + lax.axis_index("c")


_EMB_CH = 32


def _emb_gather_body(x_hbm, emb_hbm, out_hbm,
                     idx0, idx1, rows0, rows1, sem0, sem1):
    w = _wid()
    rows_per_w = TOK // _NW
    nch = rows_per_w // _EMB_CH
    idxs = (idx0, idx1)
    rows = (rows0, rows1)
    sems = (sem0, sem1)
    prev = None
    for k in range(nch):
        base = w * rows_per_w + k * _EMB_CH
        pltpu.sync_copy(x_hbm.at[pl.ds(base, _EMB_CH)], idxs[k % 2])
        cur = pltpu.async_copy(emb_hbm.at[idxs[k % 2]], rows[k % 2],
                               sems[k % 2])
        if prev is not None:
            prev.wait()
            pb = w * rows_per_w + (k - 1) * _EMB_CH
            pltpu.sync_copy(rows[(k - 1) % 2], out_hbm.at[pl.ds(pb, _EMB_CH)])
        prev = cur
    prev.wait()
    lb = w * rows_per_w + (nch - 1) * _EMB_CH
    pltpu.sync_copy(rows[(nch - 1) % 2], out_hbm.at[pl.ds(lb, _EMB_CH)])


def _emb_gather(x_flat, emb):
    return pl.kernel(
        _emb_gather_body,
        out_type=jax.ShapeDtypeStruct((TOK, DIM), jnp.float32),
        mesh=_sc_mesh(),
        compiler_params=pltpu.CompilerParams(needs_layout_passes=False),
        scratch_types=[
            pltpu.VMEM((_EMB_CH,), jnp.int32),
            pltpu.VMEM((_EMB_CH,), jnp.int32),
            pltpu.VMEM((_EMB_CH, DIM), jnp.float32),
            pltpu.VMEM((_EMB_CH, DIM), jnp.float32),
            pltpu.SemaphoreType.DMA,
            pltpu.SemaphoreType.DMA,
        ],
    )(x_flat, emb)


_BLK0 = 512


def _mlp0_body(h_ref, w1_ref, w2_ref, g_ref, b_ref,
               hout_ref, mask_ref):
    h = h_ref[...]
    mean = jnp.mean(h, axis=-1, keepdims=True)
    var = jnp.mean((h - mean) ** 2, axis=-1, keepdims=True)
    ln = (h - mean) / jnp.sqrt(var + 1e-5) * g_ref[...] + b_ref[...]
    mid = jax.nn.gelu(jnp.dot(ln, w1_ref[...],
                              preferred_element_type=jnp.float32))
    hout = h + jnp.dot(mid, w2_ref[...], preferred_element_type=jnp.float32)
    hout_ref[...] = hout
    num = jnp.sum(h * hout, axis=-1, keepdims=True)
    den = (jnp.sqrt(jnp.sum(h * h, axis=-1, keepdims=True))
           * jnp.sqrt(jnp.sum(hout * hout, axis=-1, keepdims=True)) + 1e-8)
    cos = num / den
    mask_ref[...] = (cos < THRESH).astype(jnp.int32)


def _mlp0(h, W1, W2, g, b):
    return pl.pallas_call(
        _mlp0_body,
        grid=(TOK // _BLK0,),
        in_specs=[
            pl.BlockSpec((_BLK0, DIM), lambda i: (i, 0)),
            pl.BlockSpec((DIM, DFF), lambda i: (0, 0)),
            pl.BlockSpec((DFF, DIM), lambda i: (0, 0)),
            pl.BlockSpec((1, DIM), lambda i: (0, 0)),
            pl.BlockSpec((1, DIM), lambda i: (0, 0)),
        ],
        out_specs=[
            pl.BlockSpec((_BLK0, DIM), lambda i: (i, 0)),
            pl.BlockSpec((_BLK0, 1), lambda i: (i, 0)),
        ],
        out_shape=[
            jax.ShapeDtypeStruct((TOK, DIM), jnp.float32),
            jax.ShapeDtypeStruct((TOK, 1), jnp.int32),
        ],
        compiler_params=pltpu.CompilerParams(
            dimension_semantics=("arbitrary",),
            vmem_limit_bytes=100 * 1024 * 1024),
    )(h, W1, W2, g, b)


def _compact_body(mask_hbm, idx_hbm, cnt_hbm, mask_v, idx_v, cnt_v):
    @pl.when(_wid() == 0)
    def _():
        pltpu.sync_copy(mask_hbm, mask_v)

        def step(i, off):
            mv = mask_v[pl.ds(i * 16, 16)]
            ids = lax.iota(jnp.int32, 16) + i * 16
            cnt = jnp.sum(mv, axis=0)
            plsc.store_compressed(idx_v.at[pl.ds(off, 16)], ids, mask=mv != 0)
            return off + cnt

        count = lax.fori_loop(0, TOK // 16, step, jnp.int32(0))
        idx_v[pl.ds(count, 16)] = jnp.zeros((16,), jnp.int32)
        cnt_v[...] = jnp.full((16,), count, dtype=jnp.int32)
        pltpu.sync_copy(idx_v, idx_hbm)
        pltpu.sync_copy(cnt_v, cnt_hbm)


def _compact(mask_flat):
    return pl.kernel(
        _compact_body,
        out_type=[
            jax.ShapeDtypeStruct((IDXPAD,), jnp.int32),
            jax.ShapeDtypeStruct((16,), jnp.int32),
        ],
        mesh=_sc_mesh(),
        compiler_params=pltpu.CompilerParams(needs_layout_passes=False),
        scratch_types=[
            pltpu.VMEM((TOK,), jnp.int32),
            pltpu.VMEM((IDXPAD,), jnp.int32),
            pltpu.VMEM((16,), jnp.int32),
        ],
    )(mask_flat)


def _gather_rows_body(idx_hbm, cnt_hbm, src_hbm, out_hbm,
                      cnt_v, idx_v, rows_v, sem):
    w = _wid()
    pltpu.sync_copy(cnt_hbm, cnt_v)
    count = jnp.max(cnt_v[...], axis=0)
    nchunks = (count + 15) // 16
    n_mine = jnp.maximum(0, (nchunks - w + _NW - 1) // _NW)

    def step(k, _):
        base = (w + k * _NW) * 16
        pltpu.sync_copy(idx_hbm.at[pl.ds(base, 16)], idx_v)
        pltpu.async_copy(src_hbm.at[idx_v], rows_v, sem).wait()
        pltpu.sync_copy(rows_v, out_hbm.at[pl.ds(base, 16)])
        return 0

    lax.fori_loop(0, n_mine, step, 0)


def _gather_rows(idx, cnt, src):
    return pl.kernel(
        _gather_rows_body,
        out_type=jax.ShapeDtypeStruct((TOK, DIM), jnp.float32),
        mesh=_sc_mesh(),
        compiler_params=pltpu.CompilerParams(needs_layout_passes=False),
        scratch_types=[
            pltpu.VMEM((16,), jnp.int32),
            pltpu.VMEM((16,), jnp.int32),
            pltpu.VMEM((16, DIM), jnp.float32),
            pltpu.SemaphoreType.DMA,
        ],
    )(idx, cnt, src)


_BLK1 = 256


def _mlp1_body(hprev_ref, hc_ref, w1_ref, w2_ref, g_ref, b_ref,
               cnt_ref, idx_ref, hfinal_ref,
               w1_v, w2_v, hb_v, ob_v, sem_w, sem_h, sem_s):
    del hprev_ref
    count = cnt_ref[0]
    nblk = (count + _BLK1 - 1) // _BLK1

    @pl.when(nblk > 0)
    def _():
        cw1 = pltpu.make_async_copy(w1_ref, w1_v, sem_w)
        cw1.start()
        cw2 = pltpu.make_async_copy(w2_ref, w2_v, sem_w)
        cw2.start()
        cw1.wait()
        cw2.wait()
        g = g_ref[...]
        b = b_ref[...]

        def blk(bi, _):
            base = bi * _BLK1
            ch = pltpu.make_async_copy(hc_ref.at[pl.ds(base, _BLK1)], hb_v,
                                       sem_h)
            ch.start()
            ch.wait()
            h = hb_v[...]
            mean = jnp.mean(h, axis=-1, keepdims=True)
            var = jnp.mean((h - mean) ** 2, axis=-1, keepdims=True)
            ln = (h - mean) / jnp.sqrt(var + 1e-5) * g + b
            mid = jax.nn.gelu(jnp.dot(ln, w1_v[...],
                                      preferred_element_type=jnp.float32))
            ob_v[...] = h + jnp.dot(mid, w2_v[...],
                                    preferred_element_type=jnp.float32)
            rows = jnp.minimum(count - base, _BLK1)

            def row(r, _):
                dst = idx_ref[base + r]
                c = pltpu.make_async_copy(ob_v.at[pl.ds(r, 1)],
                                          hfinal_ref.at[pl.ds(dst, 1)],
                                          sem_s)
                c.start()
                c.wait()
                return 0

            lax.fori_loop(0, rows, row, 0)
            return 0

        lax.fori_loop(0, nblk, blk, 0)


def _mlp1_scatter(h_prev, h_c, W1, W2, g, b, cnt, idx):
    return pl.pallas_call(
        _mlp1_body,
        in_specs=[
            pl.BlockSpec(memory_space=pl.ANY),
            pl.BlockSpec(memory_space=pl.ANY),
            pl.BlockSpec(memory_space=pl.ANY),
            pl.BlockSpec(memory_space=pl.ANY),
            pl.BlockSpec(memory_space=pltpu.VMEM),
            pl.BlockSpec(memory_space=pltpu.VMEM),
            pl.BlockSpec(memory_space=pltpu.SMEM),
            pl.BlockSpec(memory_space=pltpu.SMEM),
        ],
        out_specs=pl.BlockSpec(memory_space=pl.ANY),
        out_shape=jax.ShapeDtypeStruct((TOK, DIM), jnp.float32),
        scratch_shapes=[
            pltpu.VMEM((DIM, DFF), jnp.float32),
            pltpu.VMEM((DFF, DIM), jnp.float32),
            pltpu.VMEM((_BLK1, DIM), jnp.float32),
            pltpu.VMEM((_BLK1, DIM), jnp.float32),
            pltpu.SemaphoreType.DMA,
            pltpu.SemaphoreType.DMA,
            pltpu.SemaphoreType.DMA,
        ],
        input_output_aliases={0: 0},
    )(h_prev, h_c, W1, W2, g, b, cnt, idx)


_VBLK = 1024


def _logits_body(h_ref, w_ref, out_ref):
    out_ref[...] = lax.dot_general(
        h_ref[...], w_ref[...],
        dimension_numbers=(((1,), (1,)), ((), ())),
        preferred_element_type=jnp.float32)


def _logits(h, W_out):
    return pl.pallas_call(
        _logits_body,
        grid=(VOCAB // _VBLK,),
        in_specs=[
            pl.BlockSpec((TOK, DIM), lambda i: (0, 0)),
            pl.BlockSpec((_VBLK, DIM), lambda i: (i, 0)),
        ],
        out_specs=pl.BlockSpec((TOK, _VBLK), lambda i: (0, i)),
        out_shape=jax.ShapeDtypeStruct((TOK, VOCAB), jnp.float32),
        compiler_params=pltpu.CompilerParams(
            dimension_semantics=("arbitrary",),
            vmem_limit_bytes=100 * 1024 * 1024),
    )(h, W_out)


def kernel(x, emb, W_out, W1_0, W2_0, g0, b0, W1_1, W2_1, g1, b1):
    batch, seq = x.shape
    x_flat = x.reshape(-1).astype(jnp.int32)

    h = _emb_gather(x_flat, emb)
    h_out0, mask_col = _mlp0(h, W1_0, W2_0,
                             g0.reshape(1, DIM), b0.reshape(1, DIM))
    mask_flat = mask_col.reshape(TOK)
    idx, cntv = _compact(mask_flat)
    h_c = _gather_rows(idx, cntv, h_out0)
    h_final = _mlp1_scatter(h_out0, h_c, W1_1, W2_1,
                            g1.reshape(1, DIM), b1.reshape(1, DIM),
                            cntv, idx)
    logits = _logits(h_final, W_out)

    count1 = cntv[0]
    exit_counts = jnp.stack([TOK - count1, count1]).astype(jnp.int32)
    shallow_ratio = exit_counts[0] / (batch * seq)
    cumulative_layers = jnp.arange(1, 3)
    total_layers = jnp.sum(exit_counts * cumulative_layers)
    compute_cost = total_layers / ((batch * seq) * 2)
    return (logits.reshape(batch, seq, VOCAB), exit_counts,
            shallow_ratio, compute_cost)


# speedup vs baseline: 1.0935x; 1.0008x over previous
import functools

import jax
import jax.numpy as jnp
from jax import lax
from jax.experimental import pallas as pl
from jax.experimental.pallas import tpu as pltpu
from jax.experimental.pallas import tpu_sc as plsc

VOCAB = 8192
DIM = 1024
DFF = 4096
TOK = 4096
THRESH = 0.8
IDXPAD = TOK + 16

_NC, _NS = 2, 16
_NW = _NC * _NS

_sc_mesh = lambda: plsc.VectorSubcoreMesh(
    core_axis_name="c", subcore_axis_name="s")


def _wid():
    return lax.axis_index("s") * _NC + lax.axis_index("c")


_EMB_CH = 32


def _emb_gather_body(x_hbm, emb_hbm, out_hbm,
                     idx0, idx1, rows0, rows1, sem0, sem1):
    w = _wid()
    rows_per_w = TOK // _NW
    nch = rows_per_w // _EMB_CH
    idxs = (idx0, idx1)
    rows = (rows0, rows1)
    sems = (sem0, sem1)
    prev = None
    for k in range(nch):
        base = w * rows_per_w + k * _EMB_CH
        pltpu.sync_copy(x_hbm.at[pl.ds(base, _EMB_CH)], idxs[k % 2])
        cur = pltpu.async_copy(emb_hbm.at[idxs[k % 2]], rows[k % 2],
                               sems[k % 2])
        if prev is not None:
            prev.wait()
            pb = w * rows_per_w + (k - 1) * _EMB_CH
            pltpu.sync_copy(rows[(k - 1) % 2], out_hbm.at[pl.ds(pb, _EMB_CH)])
        prev = cur
    prev.wait()
    lb = w * rows_per_w + (nch - 1) * _EMB_CH
    pltpu.sync_copy(rows[(nch - 1) % 2], out_hbm.at[pl.ds(lb, _EMB_CH)])


def _emb_gather(x_flat, emb):
    return pl.kernel(
        _emb_gather_body,
        out_type=jax.ShapeDtypeStruct((TOK, DIM), jnp.float32),
        mesh=_sc_mesh(),
        compiler_params=pltpu.CompilerParams(needs_layout_passes=False),
        scratch_types=[
            pltpu.VMEM((_EMB_CH,), jnp.int32),
            pltpu.VMEM((_EMB_CH,), jnp.int32),
            pltpu.VMEM((_EMB_CH, DIM), jnp.float32),
            pltpu.VMEM((_EMB_CH, DIM), jnp.float32),
            pltpu.SemaphoreType.DMA,
            pltpu.SemaphoreType.DMA,
        ],
    )(x_flat, emb)


_BLK0 = 512


def _mlp0_body(h_ref, w1_ref, w2_ref, g_ref, b_ref,
               hout_ref, mask_ref):
    h = h_ref[...]
    mean = jnp.mean(h, axis=-1, keepdims=True)
    var = jnp.mean((h - mean) ** 2, axis=-1, keepdims=True)
    ln = (h - mean) / jnp.sqrt(var + 1e-5) * g_ref[...] + b_ref[...]
    mid = jax.nn.gelu(jnp.dot(ln, w1_ref[...],
                              preferred_element_type=jnp.float32))
    hout = h + jnp.dot(mid, w2_ref[...], preferred_element_type=jnp.float32)
    hout_ref[...] = hout
    num = jnp.sum(h * hout, axis=-1, keepdims=True)
    den = (jnp.sqrt(jnp.sum(h * h, axis=-1, keepdims=True))
           * jnp.sqrt(jnp.sum(hout * hout, axis=-1, keepdims=True)) + 1e-8)
    cos = num / den
    mask_ref[...] = (cos < THRESH).astype(jnp.int32)


def _mlp0(h, W1, W2, g, b):
    return pl.pallas_call(
        _mlp0_body,
        grid=(TOK // _BLK0,),
        in_specs=[
            pl.BlockSpec((_BLK0, DIM), lambda i: (i, 0)),
            pl.BlockSpec((DIM, DFF), lambda i: (0, 0)),
            pl.BlockSpec((DFF, DIM), lambda i: (0, 0)),
            pl.BlockSpec((1, DIM), lambda i: (0, 0)),
            pl.BlockSpec((1, DIM), lambda i: (0, 0)),
        ],
        out_specs=[
            pl.BlockSpec((_BLK0, DIM), lambda i: (i, 0)),
            pl.BlockSpec((_BLK0, 1), lambda i: (i, 0)),
        ],
        out_shape=[
            jax.ShapeDtypeStruct((TOK, DIM), jnp.float32),
            jax.ShapeDtypeStruct((TOK, 1), jnp.int32),
        ],
        compiler_params=pltpu.CompilerParams(
            dimension_semantics=("arbitrary",),
            vmem_limit_bytes=100 * 1024 * 1024),
    )(h, W1, W2, g, b)


def _compact_body(mask_hbm, idx_hbm, cnt_hbm, mask_v, idx_v, cnt_v):
    @pl.when(_wid() == 0)
    def _():
        pltpu.sync_copy(mask_hbm, mask_v)

        def step(i, off):
            mv = mask_v[pl.ds(i * 16, 16)]
            ids = lax.iota(jnp.int32, 16) + i * 16
            cnt = jnp.sum(mv, axis=0)
            plsc.store_compressed(idx_v.at[pl.ds(off, 16)], ids, mask=mv != 0)
            return off + cnt

        count = lax.fori_loop(0, TOK // 16, step, jnp.int32(0))
        idx_v[pl.ds(count, 16)] = jnp.zeros((16,), jnp.int32)
        cnt_v[...] = jnp.full((16,), count, dtype=jnp.int32)
        pltpu.sync_copy(idx_v, idx_hbm)
        pltpu.sync_copy(cnt_v, cnt_hbm)


def _compact(mask_flat):
    return pl.kernel(
        _compact_body,
        out_type=[
            jax.ShapeDtypeStruct((IDXPAD,), jnp.int32),
            jax.ShapeDtypeStruct((16,), jnp.int32),
        ],
        mesh=_sc_mesh(),
        compiler_params=pltpu.CompilerParams(needs_layout_passes=False),
        scratch_types=[
            pltpu.VMEM((TOK,), jnp.int32),
            pltpu.VMEM((IDXPAD,), jnp.int32),
            pltpu.VMEM((16,), jnp.int32),
        ],
    )(mask_flat)


def _gather_rows_body(idx_hbm, cnt_hbm, src_hbm, out_hbm,
                      cnt_v, idx_v, rows_v, sem):
    w = _wid()
    pltpu.sync_copy(cnt_hbm, cnt_v)
    count = jnp.max(cnt_v[...], axis=0)
    nchunks = (count + 15) // 16
    n_mine = jnp.maximum(0, (nchunks - w + _NW - 1) // _NW)

    def step(k, _):
        base = (w + k * _NW) * 16
        pltpu.sync_copy(idx_hbm.at[pl.ds(base, 16)], idx_v)
        pltpu.async_copy(src_hbm.at[idx_v], rows_v, sem).wait()
        pltpu.sync_copy(rows_v, out_hbm.at[pl.ds(base, 16)])
        return 0

    lax.fori_loop(0, n_mine, step, 0)


def _gather_rows(idx, cnt, src):
    return pl.kernel(
        _gather_rows_body,
        out_type=jax.ShapeDtypeStruct((TOK, DIM), jnp.float32),
        mesh=_sc_mesh(),
        compiler_params=pltpu.CompilerParams(needs_layout_passes=False),
        scratch_types=[
            pltpu.VMEM((16,), jnp.int32),
            pltpu.VMEM((16,), jnp.int32),
            pltpu.VMEM((16, DIM), jnp.float32),
            pltpu.SemaphoreType.DMA,
        ],
    )(idx, cnt, src)


_BLK1 = 128
_CH5 = 1024


def _mlp1_body(lg0_ref, hc_ref, w1_ref, w2_ref, wout_ref, g_ref, b_ref,
               cnt_ref, idx_ref, lg_ref,
               w1_v, w2_v, woutb_v, stage_v, hb_v, lrow_v,
               sem_w, sem_h, sem_s):
    del lg0_ref
    count = cnt_ref[0]
    nblk = (count + _BLK1 - 1) // _BLK1

    @pl.when(nblk > 0)
    def _():
        cw1 = pltpu.make_async_copy(w1_ref, w1_v, sem_w)
        cw1.start()
        cw2 = pltpu.make_async_copy(w2_ref, w2_v, sem_w)
        cw2.start()
        cw1.wait()
        cw2.wait()
        for c in range(VOCAB // _CH5):
            cp = pltpu.make_async_copy(
                wout_ref.at[pl.ds(c * _CH5, _CH5)], stage_v, sem_w)
            cp.start()
            cp.wait()
            woutb_v[pl.ds(c * _CH5, _CH5)] = stage_v[...].astype(
                jnp.bfloat16)
        g = g_ref[...]
        b = b_ref[...]

        def blk(bi, _):
            base = bi * _BLK1
            ch = pltpu.make_async_copy(hc_ref.at[pl.ds(base, _BLK1)], hb_v,
                                       sem_h)
            ch.start()
            ch.wait()
            h = hb_v[...]
            mean = jnp.mean(h, axis=-1, keepdims=True)
            var = jnp.mean((h - mean) ** 2, axis=-1, keepdims=True)
            ln = (h - mean) / jnp.sqrt(var + 1e-5) * g + b
            mid = jax.nn.gelu(jnp.dot(ln.astype(jnp.bfloat16),
                                      w1_v[...].astype(jnp.bfloat16),
                                      preferred_element_type=jnp.float32))
            hout = h + jnp.dot(mid.astype(jnp.bfloat16),
                               w2_v[...].astype(jnp.bfloat16),
                               preferred_element_type=jnp.float32)
            lrow_v[...] = lax.dot_general(
                hout.astype(jnp.bfloat16), woutb_v[...],
                dimension_numbers=(((1,), (1,)), ((), ())),
                preferred_element_type=jnp.float32)
            rows = jnp.minimum(count - base, _BLK1)

            def row(r, _):
                dst = idx_ref[base + r]
                c = pltpu.make_async_copy(lrow_v.at[pl.ds(r, 1)],
                                          lg_ref.at[pl.ds(dst, 1)],
                                          sem_s)
                c.start()
                c.wait()
                return 0

            lax.fori_loop(0, rows, row, 0)
            return 0

        lax.fori_loop(0, nblk, blk, 0)


def _mlp1_scatter(logits0, h_c, W1, W2, Wout, g, b, cnt, idx):
    return pl.pallas_call(
        _mlp1_body,
        in_specs=[
            pl.BlockSpec(memory_space=pl.ANY),
            pl.BlockSpec(memory_space=pl.ANY),
            pl.BlockSpec(memory_space=pl.ANY),
            pl.BlockSpec(memory_space=pl.ANY),
            pl.BlockSpec(memory_space=pl.ANY),
            pl.BlockSpec(memory_space=pltpu.VMEM),
            pl.BlockSpec(memory_space=pltpu.VMEM),
            pl.BlockSpec(memory_space=pltpu.SMEM),
            pl.BlockSpec(memory_space=pltpu.SMEM),
        ],
        out_specs=pl.BlockSpec(memory_space=pl.ANY),
        out_shape=jax.ShapeDtypeStruct((TOK, VOCAB), jnp.float32),
        scratch_shapes=[
            pltpu.VMEM((DIM, DFF), jnp.float32),
            pltpu.VMEM((DFF, DIM), jnp.float32),
            pltpu.VMEM((VOCAB, DIM), jnp.bfloat16),
            pltpu.VMEM((_CH5, DIM), jnp.float32),
            pltpu.VMEM((_BLK1, DIM), jnp.float32),
            pltpu.VMEM((_BLK1, VOCAB), jnp.float32),
            pltpu.SemaphoreType.DMA,
            pltpu.SemaphoreType.DMA,
            pltpu.SemaphoreType.DMA,
        ],
        input_output_aliases={0: 0},
        compiler_params=pltpu.CompilerParams(
            vmem_limit_bytes=63 * 1024 * 1024),
    )(logits0, h_c, W1, W2, Wout, g, b, cnt, idx)


_VBLK = 1024


def _logits_body(h_ref, w_ref, out_ref):
    out_ref[...] = lax.dot_general(
        h_ref[...], w_ref[...],
        dimension_numbers=(((1,), (1,)), ((), ())),
        preferred_element_type=jnp.float32)


def _logits(h, W_out):
    return pl.pallas_call(
        _logits_body,
        grid=(VOCAB // _VBLK,),
        in_specs=[
            pl.BlockSpec((TOK, DIM), lambda i: (0, 0)),
            pl.BlockSpec((_VBLK, DIM), lambda i: (i, 0)),
        ],
        out_specs=pl.BlockSpec((TOK, _VBLK), lambda i: (0, i)),
        out_shape=jax.ShapeDtypeStruct((TOK, VOCAB), jnp.float32),
        compiler_params=pltpu.CompilerParams(
            dimension_semantics=("arbitrary",),
            vmem_limit_bytes=100 * 1024 * 1024),
    )(h, W_out)


def kernel(x, emb, W_out, W1_0, W2_0, g0, b0, W1_1, W2_1, g1, b1):
    batch, seq = x.shape
    x_flat = x.reshape(-1).astype(jnp.int32)

    h = _emb_gather(x_flat, emb)
    h_out0, mask_col = _mlp0(h, W1_0, W2_0,
                             g0.reshape(1, DIM), b0.reshape(1, DIM))
    mask_flat = mask_col.reshape(TOK)
    logits0 = _logits(h_out0, W_out)
    idx, cntv = _compact(mask_flat)
    h_c = _gather_rows(idx, cntv, h_out0)
    logits = _mlp1_scatter(logits0, h_c, W1_1, W2_1, W_out,
                           g1.reshape(1, DIM), b1.reshape(1, DIM),
                           cntv, idx)

    count1 = cntv[0]
    exit_counts = jnp.stack([TOK - count1, count1]).astype(jnp.int32)
    shallow_ratio = exit_counts[0] / (batch * seq)
    cumulative_layers = jnp.arange(1, 3)
    total_layers = jnp.sum(exit_counts * cumulative_layers)
    compute_cost = total_layers / ((batch * seq) * 2)
    return (logits.reshape(batch, seq, VOCAB), exit_counts,
            shallow_ratio, compute_cost)
